# baseline (speedup 1.0000x reference)
import jax
import jax.numpy as jnp
from jax import lax
from jax.experimental import pallas as pl
from jax.experimental.pallas import tpu as pltpu
from jax.experimental.pallas import tpu_sc as plsc

D = 64
PHRASE = 4
LANES = 16
C = 128
VC = 4096
HB = VC // 2


def _tc_repack(ut, vt, V):
    NB = (V + VC - 1) // VC
    LB = (V + HB - 1) // HB - 1

    def body(ut_t, ut_b, vt_t, vt_b, uo_ref, vo_ref):
        for src_t, src_b, dst in ((ut_t, ut_b, uo_ref), (vt_t, vt_b, vo_ref)):
            tt = jnp.transpose(src_t[...].astype(jnp.bfloat16), (1, 0))
            tb = jnp.transpose(src_b[...].astype(jnp.bfloat16), (1, 0))
            dst[...] = jnp.concatenate([tt, tb], axis=1)

    return pl.pallas_call(
        body,
        grid=(NB,),
        in_specs=[
            pl.BlockSpec((D, HB), lambda k: (0, 2 * k)),
            pl.BlockSpec((D, HB), lambda k: (0, jnp.minimum(2 * k + 1, LB))),
            pl.BlockSpec((D, HB), lambda k: (0, 2 * k)),
            pl.BlockSpec((D, HB), lambda k: (0, jnp.minimum(2 * k + 1, LB))),
        ],
        out_specs=[
            pl.BlockSpec((HB, 2 * D), lambda k: (k, 0)),
            pl.BlockSpec((HB, 2 * D), lambda k: (k, 0)),
        ],
        out_shape=[
            jax.ShapeDtypeStruct((NB * HB, 2 * D), jnp.bfloat16),
            jax.ShapeDtypeStruct((NB * HB, 2 * D), jnp.bfloat16),
        ],
    )(ut, ut, vt, vt)


def _sc_gather_pool(pu_t, pv_t, nv_t, u_r, v_r, B, NEG):
    info = plsc.get_sparse_core_info()
    NC, NS = info.num_cores, info.num_subcores
    W = NC * NS
    SB = B // W
    n_chunks = SB // C
    mesh = plsc.VectorSubcoreMesh(core_axis_name="c", subcore_axis_name="s")

    def body(pu_ref, pv_ref, nv_ref, u_ref, v_ref, eu_ref, evng_ref,
             idx_s, row_s, rows_s, emb_s, sem):
        wid = lax.axis_index("s") * NC + lax.axis_index("c")
        base = wid * SB

        def do_chunk(table_ref, idx_hbm, idx_row, src_base, out_ref, out_b2):
            for p in range(PHRASE):
                src = pl.multiple_of((idx_row + p) * B + src_base, 128)
                pltpu.sync_copy(idx_hbm.at[pl.ds(src, C)], idx_s.at[p])
            for p in range(PHRASE):
                for k in range(C // LANES):
                    v = idx_s[p, pl.ds(k * LANES, LANES)]
                    row_s[p, pl.ds(k * LANES, LANES)] = (
                        ((v >> 12) << 11) | (v & (HB - 1)))
            hs = [pltpu.async_copy(table_ref.at[row_s.at[p]],
                                   rows_s.at[pl.ds(p * C, C)], sem)
                  for p in range(PHRASE)]
            for h in hs:
                h.wait()

            def pool(g, carry):
                offv = [((idx_s[p, pl.ds(g * LANES, LANES)] >> 5) & 64)
                        for p in range(PHRASE)]
                for l in range(LANES):
                    i = g * LANES + l
                    i2 = g * (LANES // 2) + l // 2
                    e = l % 2
                    hi = [offv[p][l] > 0 for p in range(PHRASE)]
                    for k in range(D // 32):
                        acc = None
                        for p in range(PHRASE):
                            j = p * C + i
                            v = jnp.where(hi[p],
                                          rows_s[j, pl.ds(D + k * 32, 32)],
                                          rows_s[j, pl.ds(k * 32, 32)])
                            acc = v if acc is None else acc + v
                        emb_s[i2, pl.ds(e * D + k * 32, 32)] = acc
                return carry
            lax.fori_loop(0, C // LANES, pool, 0)
            pltpu.sync_copy(emb_s,
                            out_ref.at[pl.ds(pl.multiple_of(out_b2, 64),
                                             C // 2)])

        def u_pass(c, carry):
            do_chunk(u_ref, pu_ref, 0, base + c * C, eu_ref,
                     (base + c * C) // 2)
            return carry
        lax.fori_loop(0, n_chunks, u_pass, 0)

        def v_pass(t, carry):
            c = t % n_chunks
            g = t // n_chunks
            src_base = base + c * C
            out_b2 = (g * B + base + c * C) // 2

            @pl.when(g == 0)
            def _():
                do_chunk(v_ref, pv_ref, 0, src_base, evng_ref, out_b2)

            @pl.when(g > 0)
            def _():
                do_chunk(v_ref, nv_ref, (g - 1) * PHRASE, src_base,
                         evng_ref, out_b2)
            return carry
        lax.fori_loop(0, (1 + NEG) * n_chunks, v_pass, 0)

    f = pl.kernel(
        body,
        out_type=(
            jax.ShapeDtypeStruct((B // 2, 2 * D), jnp.bfloat16),
            jax.ShapeDtypeStruct(((1 + NEG) * B // 2, 2 * D), jnp.bfloat16),
        ),
        mesh=mesh,
        scratch_types=(
            pltpu.VMEM((PHRASE, C), jnp.int32),
            pltpu.VMEM((PHRASE, C), jnp.int32),
            pltpu.VMEM((PHRASE * C, 2 * D), jnp.bfloat16),
            pltpu.VMEM((C // 2, 2 * D), jnp.bfloat16),
            pltpu.SemaphoreType.DMA,
        ),
        compiler_params=pltpu.CompilerParams(use_tc_tiling_on_sc=False),
    )
    return f(pu_t, pv_t, nv_t, u_r, v_r)


def _tc_loss(eu2, evng2, B, NEG):
    TB2 = 512
    grid = (B // 2) // TB2
    nblk = (B // 2) // TB2
    inv = 1.0 / (PHRASE * PHRASE)

    def body(eu_ref, ev_ref, *rest):
        ng_refs, out_ref = rest[:-1], rest[-1]
        i = pl.program_id(0)
        eu = eu_ref[...].astype(jnp.float32)
        ev = ev_ref[...].astype(jnp.float32)

        def lsig(x):
            return jnp.minimum(x, 0.0) - jnp.log1p(jnp.exp(-jnp.abs(x)))

        def pair_scores(a, b):
            s0 = jnp.sum(a[:, :D] * b[:, :D], axis=1) * inv
            s1 = jnp.sum(a[:, D:] * b[:, D:], axis=1) * inv
            return s0, s1

        s0, s1 = pair_scores(eu, ev)
        tot = jnp.sum(lsig(s0)) + jnp.sum(lsig(s1))
        for ng_ref in ng_refs:
            n0, n1 = pair_scores(eu, ng_ref[...].astype(jnp.float32))
            tot = tot + jnp.sum(lsig(-n0)) + jnp.sum(lsig(-n1))

        @pl.when(i == 0)
        def _():
            out_ref[0, 0] = 0.0
        out_ref[0, 0] += tot

    def ng_spec(n):
        return pl.BlockSpec((TB2, 2 * D), lambda i, n=n: (nblk * (1 + n) + i, 0))

    out = pl.pallas_call(
        body,
        grid=(grid,),
        in_specs=[pl.BlockSpec((TB2, 2 * D), lambda i: (i, 0)),
                  pl.BlockSpec((TB2, 2 * D), lambda i: (i, 0))]
                 + [ng_spec(n) for n in range(NEG)],
        out_specs=pl.BlockSpec(memory_space=pltpu.SMEM),
        out_shape=jax.ShapeDtypeStruct((1, 1), jnp.float32),
        compiler_params=pltpu.CompilerParams(
            dimension_semantics=("arbitrary",)),
    )(eu2, evng2, *([evng2] * NEG))
    return out[0, 0]


def kernel(pos_u, pos_v, neg_v, batch_size, u_weight, v_weight):
    B, P_ = pos_u.shape
    V = u_weight.shape[0]
    NEG = neg_v.shape[0] // B
    u_r, v_r = _tc_repack(u_weight.T, v_weight.T, V)
    pu_t = pos_u.T.reshape(-1)
    pv_t = pos_v.T.reshape(-1)
    nv_t = neg_v.reshape(B, NEG, P_).transpose(1, 2, 0).reshape(-1)
    eu2, evng2 = _sc_gather_pool(pu_t, pv_t, nv_t, u_r, v_r, B, NEG)
    total = _tc_loss(eu2, evng2, B, NEG)
    return -total / jnp.asarray(batch_size, jnp.float32)


# speedup vs baseline: 1.4646x; 1.4646x over previous
import jax
import jax.numpy as jnp
from jax import lax
from jax.experimental import pallas as pl
from jax.experimental.pallas import tpu as pltpu
from jax.experimental.pallas import tpu_sc as plsc

D = 64
PHRASE = 4
LANES = 16
C = 128
VC = 4096
HB = VC // 2


def _tc_repack(t, V):
    NB = (V + VC - 1) // VC
    LB = (V + HB - 1) // HB - 1

    def body(src_t, src_b, dst):
        tt = jnp.transpose(src_t[...], (1, 0))
        tb = jnp.transpose(src_b[...], (1, 0))
        dst[...] = jnp.concatenate([tt, tb], axis=1)

    return pl.pallas_call(
        body,
        grid=(NB,),
        in_specs=[
            pl.BlockSpec((D, HB), lambda k: (0, 2 * k)),
            pl.BlockSpec((D, HB), lambda k: (0, jnp.minimum(2 * k + 1, LB))),
        ],
        out_specs=pl.BlockSpec((HB, 2 * D), lambda k: (k, 0)),
        out_shape=jax.ShapeDtypeStruct((NB * HB, 2 * D), jnp.float32),
    )(t, t)


_SC_SCRATCH = (
    pltpu.VMEM((PHRASE, C), jnp.int32),
    pltpu.VMEM((PHRASE, C), jnp.int32),
    pltpu.VMEM((PHRASE * C, 2 * D), jnp.float32),
    pltpu.VMEM((C // 2, 2 * D), jnp.float32),
    pltpu.SemaphoreType.DMA,
)


def _make_do_chunk(B, idx_s, row_s, rows_s, emb_s, sem):
    def do_chunk(table_ref, idx_hbm, idx_row, src_base, out_ref, out_b2):
            for p in range(PHRASE):
                src = pl.multiple_of((idx_row + p) * B + src_base, 128)
                pltpu.sync_copy(idx_hbm.at[pl.ds(src, C)], idx_s.at[p])
            for p in range(PHRASE):
                for k in range(C // LANES):
                    v = idx_s[p, pl.ds(k * LANES, LANES)]
                    row_s[p, pl.ds(k * LANES, LANES)] = (
                        ((v >> 12) << 11) | (v & (HB - 1)))
            hs = [pltpu.async_copy(table_ref.at[row_s.at[p]],
                                   rows_s.at[pl.ds(p * C, C)], sem)
                  for p in range(PHRASE)]
            for h in hs:
                h.wait()

            def pool(g, carry):
                offv = [((idx_s[p, pl.ds(g * LANES, LANES)] >> 5) & 64)
                        for p in range(PHRASE)]
                for l in range(LANES):
                    i = g * LANES + l
                    i2 = g * (LANES // 2) + l // 2
                    e = l % 2
                    off = [offv[p][l] for p in range(PHRASE)]
                    for d in range(D // LANES):
                        acc = (rows_s[i, pl.ds(off[0] + d * LANES, LANES)]
                               + rows_s[C + i, pl.ds(off[1] + d * LANES, LANES)]
                               + rows_s[2 * C + i, pl.ds(off[2] + d * LANES, LANES)]
                               + rows_s[3 * C + i, pl.ds(off[3] + d * LANES, LANES)])
                        emb_s[i2, pl.ds(e * D + d * LANES, LANES)] = acc
                return carry
            lax.fori_loop(0, C // LANES, pool, 0)
            pltpu.sync_copy(emb_s,
                            out_ref.at[pl.ds(pl.multiple_of(out_b2, 64),
                                             C // 2)])

    return do_chunk


def _sc_mesh_info():
    info = plsc.get_sparse_core_info()
    return info.num_cores, info.num_subcores


def _sc_gather_u(pu_t, u_r, B):
    NC, NS = _sc_mesh_info()
    SB = B // (NC * NS)
    n_chunks = SB // C
    mesh = plsc.VectorSubcoreMesh(core_axis_name="c", subcore_axis_name="s")

    def body(pu_ref, u_ref, eu_ref, idx_s, row_s, rows_s, emb_s, sem):
        wid = lax.axis_index("s") * NC + lax.axis_index("c")
        base = wid * SB
        do_chunk = _make_do_chunk(B, idx_s, row_s, rows_s, emb_s, sem)

        def u_pass(c, carry):
            do_chunk(u_ref, pu_ref, 0, base + c * C, eu_ref,
                     (base + c * C) // 2)
            return carry
        lax.fori_loop(0, n_chunks, u_pass, 0)

    f = pl.kernel(
        body,
        out_type=jax.ShapeDtypeStruct((B // 2, 2 * D), jnp.float32),
        mesh=mesh,
        scratch_types=_SC_SCRATCH,
        compiler_params=pltpu.CompilerParams(use_tc_tiling_on_sc=True),
    )
    return f(pu_t, u_r)


def _sc_gather_v(pv_t, nv_t, v_r, B, NEG):
    NC, NS = _sc_mesh_info()
    SB = B // (NC * NS)
    n_chunks = SB // C
    mesh = plsc.VectorSubcoreMesh(core_axis_name="c", subcore_axis_name="s")

    def body(pv_ref, nv_ref, v_ref, evng_ref, idx_s, row_s, rows_s, emb_s,
             sem):
        wid = lax.axis_index("s") * NC + lax.axis_index("c")
        base = wid * SB
        do_chunk = _make_do_chunk(B, idx_s, row_s, rows_s, emb_s, sem)

        def v_pass(t, carry):
            c = t % n_chunks
            g = t // n_chunks
            src_base = base + c * C
            out_b2 = (g * B + base + c * C) // 2

            @pl.when(g == 0)
            def _():
                do_chunk(v_ref, pv_ref, 0, src_base, evng_ref, out_b2)

            @pl.when(g > 0)
            def _():
                do_chunk(v_ref, nv_ref, (g - 1) * PHRASE, src_base,
                         evng_ref, out_b2)
            return carry
        lax.fori_loop(0, (1 + NEG) * n_chunks, v_pass, 0)

    f = pl.kernel(
        body,
        out_type=jax.ShapeDtypeStruct(((1 + NEG) * B // 2, 2 * D),
                                      jnp.float32),
        mesh=mesh,
        scratch_types=_SC_SCRATCH,
        compiler_params=pltpu.CompilerParams(use_tc_tiling_on_sc=True),
    )
    return f(pv_t, nv_t, v_r)


def _tc_loss(eu2, evng2, B, NEG):
    TB2 = 512
    grid = (B // 2) // TB2
    nblk = (B // 2) // TB2
    inv = 1.0 / (PHRASE * PHRASE)

    def body(eu_ref, ev_ref, *rest):
        ng_refs, out_ref = rest[:-1], rest[-1]
        i = pl.program_id(0)
        eu = eu_ref[...]
        ev = ev_ref[...]

        def lsig(x):
            return jnp.minimum(x, 0.0) - jnp.log1p(jnp.exp(-jnp.abs(x)))

        def pair_scores(a, b):
            s0 = jnp.sum(a[:, :D] * b[:, :D], axis=1) * inv
            s1 = jnp.sum(a[:, D:] * b[:, D:], axis=1) * inv
            return s0, s1

        s0, s1 = pair_scores(eu, ev)
        tot = jnp.sum(lsig(s0)) + jnp.sum(lsig(s1))
        for ng_ref in ng_refs:
            n0, n1 = pair_scores(eu, ng_ref[...])
            tot = tot + jnp.sum(lsig(-n0)) + jnp.sum(lsig(-n1))

        @pl.when(i == 0)
        def _():
            out_ref[0, 0] = 0.0
        out_ref[0, 0] += tot

    def ng_spec(n):
        return pl.BlockSpec((TB2, 2 * D), lambda i, n=n: (nblk * (1 + n) + i, 0))

    out = pl.pallas_call(
        body,
        grid=(grid,),
        in_specs=[pl.BlockSpec((TB2, 2 * D), lambda i: (i, 0)),
                  pl.BlockSpec((TB2, 2 * D), lambda i: (i, 0))]
                 + [ng_spec(n) for n in range(NEG)],
        out_specs=pl.BlockSpec(memory_space=pltpu.SMEM),
        out_shape=jax.ShapeDtypeStruct((1, 1), jnp.float32),
        compiler_params=pltpu.CompilerParams(
            dimension_semantics=("arbitrary",)),
    )(eu2, evng2, *([evng2] * NEG))
    return out[0, 0]


def kernel(pos_u, pos_v, neg_v, batch_size, u_weight, v_weight):
    B, P_ = pos_u.shape
    V = u_weight.shape[0]
    NEG = neg_v.shape[0] // B
    pu_t = pos_u.T.reshape(-1)
    pv_t = pos_v.T.reshape(-1)
    nv_t = neg_v.reshape(B, NEG, P_).transpose(1, 2, 0).reshape(-1)
    v_r = _tc_repack(v_weight.T, V)
    evng2 = _sc_gather_v(pv_t, nv_t, v_r, B, NEG)
    u_r = _tc_repack(u_weight.T, V)
    eu2 = _sc_gather_u(pu_t, u_r, B)
    total = _tc_loss(eu2, evng2, B, NEG)
    return -total / jnp.asarray(batch_size, jnp.float32)


# speedup vs baseline: 1.7742x; 1.2114x over previous
import jax
import jax.numpy as jnp
from jax import lax
from jax.experimental import pallas as pl
from jax.experimental.pallas import tpu as pltpu
from jax.experimental.pallas import tpu_sc as plsc

D = 64
PHRASE = 4
LANES = 16
C = 128
VC = 4096
HB = VC // 2


def _tc_repack(t, V):
    NB = (V + VC - 1) // VC
    LB = (V + HB - 1) // HB - 1

    def body(src_t, src_b, dst):
        tt = jnp.transpose(src_t[...], (1, 0))
        tb = jnp.transpose(src_b[...], (1, 0))
        dst[...] = jnp.concatenate([tt, tb], axis=1)

    return pl.pallas_call(
        body,
        grid=(NB,),
        in_specs=[
            pl.BlockSpec((D, HB), lambda k: (0, 2 * k)),
            pl.BlockSpec((D, HB), lambda k: (0, jnp.minimum(2 * k + 1, LB))),
        ],
        out_specs=pl.BlockSpec((HB, 2 * D), lambda k: (k, 0)),
        out_shape=jax.ShapeDtypeStruct((NB * HB, 2 * D), jnp.float32),
    )(t, t)


_SC_SCRATCH = (
    pltpu.VMEM((PHRASE, C), jnp.int32),
    pltpu.VMEM((PHRASE, C), jnp.int32),
    pltpu.VMEM((PHRASE * C, 2 * D), jnp.float32),
    pltpu.VMEM((C // 2, 2 * D), jnp.float32),
    pltpu.SemaphoreType.DMA,
)


def _make_do_chunk(B, idx_s, row_s, rows_s, emb_s, sem):
    def do_chunk(table_ref, idx_hbm, idx_row, src_base, out_ref, out_b2):
            for p in range(PHRASE):
                src = pl.multiple_of((idx_row + p) * B + src_base, 128)
                pltpu.sync_copy(idx_hbm.at[pl.ds(src, C)], idx_s.at[p])
            for p in range(PHRASE):
                for k in range(C // LANES):
                    v = idx_s[p, pl.ds(k * LANES, LANES)]
                    row_s[p, pl.ds(k * LANES, LANES)] = (
                        ((v >> 12) << 11) | (v & (HB - 1)))
            hs = [pltpu.async_copy(table_ref.at[row_s.at[p]],
                                   rows_s.at[pl.ds(p * C, C)], sem)
                  for p in range(PHRASE)]
            for h in hs:
                h.wait()

            def pool(g, carry):
                offv = [((idx_s[p, pl.ds(g * LANES, LANES)] >> 5) & 64)
                        for p in range(PHRASE)]
                for l in range(LANES):
                    i = g * LANES + l
                    i2 = g * (LANES // 2) + l // 2
                    e = l % 2
                    off = [offv[p][l] for p in range(PHRASE)]
                    for d in range(D // LANES):
                        acc = (rows_s[i, pl.ds(off[0] + d * LANES, LANES)]
                               + rows_s[C + i, pl.ds(off[1] + d * LANES, LANES)]
                               + rows_s[2 * C + i, pl.ds(off[2] + d * LANES, LANES)]
                               + rows_s[3 * C + i, pl.ds(off[3] + d * LANES, LANES)])
                        emb_s[i2, pl.ds(e * D + d * LANES, LANES)] = acc
                return carry
            lax.fori_loop(0, C // LANES, pool, 0)
            pltpu.sync_copy(emb_s,
                            out_ref.at[pl.ds(pl.multiple_of(out_b2, 64),
                                             C // 2)])

    return do_chunk


def _sc_mesh_info():
    info = plsc.get_sparse_core_info()
    return info.num_cores, info.num_subcores


def _sc_gather_u(pu_t, u_r, B):
    NC, NS = _sc_mesh_info()
    SB = B // (NC * NS)
    n_chunks = SB // C
    mesh = plsc.VectorSubcoreMesh(core_axis_name="c", subcore_axis_name="s")

    def body(pu_ref, u_ref, eu_ref, idx_s, row_s, rows_s, emb_s, sem):
        wid = lax.axis_index("s") * NC + lax.axis_index("c")
        base = wid * SB
        do_chunk = _make_do_chunk(B, idx_s, row_s, rows_s, emb_s, sem)

        def u_pass(c, carry):
            do_chunk(u_ref, pu_ref, 0, base + c * C, eu_ref,
                     (base + c * C) // 2)
            return carry
        lax.fori_loop(0, n_chunks, u_pass, 0)

    f = pl.kernel(
        body,
        out_type=jax.ShapeDtypeStruct((B // 2, 2 * D), jnp.float32),
        mesh=mesh,
        scratch_types=_SC_SCRATCH,
        compiler_params=pltpu.CompilerParams(use_tc_tiling_on_sc=True),
    )
    return f(pu_t, u_r)


def _sc_gather_v(pv_t, nv_t, v_r, B, NEG):
    NC, NS = _sc_mesh_info()
    SB = B // (NC * NS)
    n_chunks = SB // C
    mesh = plsc.VectorSubcoreMesh(core_axis_name="c", subcore_axis_name="s")

    def body(pv_ref, nv_ref, v_ref, evng_ref, idx_s, row_s, rows_s, emb_s,
             sem):
        wid = lax.axis_index("s") * NC + lax.axis_index("c")
        base = wid * SB
        do_chunk = _make_do_chunk(B, idx_s, row_s, rows_s, emb_s, sem)

        def v_pass(t, carry):
            c = t % n_chunks
            g = t // n_chunks
            src_base = base + c * C
            out_b2 = (g * B + base + c * C) // 2

            @pl.when(g == 0)
            def _():
                do_chunk(v_ref, pv_ref, 0, src_base, evng_ref, out_b2)

            @pl.when(g > 0)
            def _():
                do_chunk(v_ref, nv_ref, (g - 1) * PHRASE, src_base,
                         evng_ref, out_b2)
            return carry
        lax.fori_loop(0, (1 + NEG) * n_chunks, v_pass, 0)

    f = pl.kernel(
        body,
        out_type=jax.ShapeDtypeStruct(((1 + NEG) * B // 2, 2 * D),
                                      jnp.float32),
        mesh=mesh,
        scratch_types=_SC_SCRATCH,
        compiler_params=pltpu.CompilerParams(use_tc_tiling_on_sc=True),
    )
    return f(pv_t, nv_t, v_r)


def _tc_loss(eu2, evng2, B, NEG):
    TB2 = 512
    grid = (B // 2) // TB2
    nblk = (B // 2) // TB2
    inv = 1.0 / (PHRASE * PHRASE)

    def body(eu_ref, ev_ref, *rest):
        ng_refs, out_ref = rest[:-1], rest[-1]
        i = pl.program_id(0)
        eu = eu_ref[...]
        ev = ev_ref[...]

        def lsig(x):
            return jnp.minimum(x, 0.0) - jnp.log1p(jnp.exp(-jnp.abs(x)))

        def pair_scores(a, b):
            s0 = jnp.sum(a[:, :D] * b[:, :D], axis=1) * inv
            s1 = jnp.sum(a[:, D:] * b[:, D:], axis=1) * inv
            return s0, s1

        s0, s1 = pair_scores(eu, ev)
        tot = jnp.sum(lsig(s0)) + jnp.sum(lsig(s1))
        for ng_ref in ng_refs:
            n0, n1 = pair_scores(eu, ng_ref[...])
            tot = tot + jnp.sum(lsig(-n0)) + jnp.sum(lsig(-n1))

        @pl.when(i == 0)
        def _():
            out_ref[0, 0] = 0.0
        out_ref[0, 0] += tot

    def ng_spec(n):
        return pl.BlockSpec((TB2, 2 * D), lambda i, n=n: (nblk * (1 + n) + i, 0))

    out = pl.pallas_call(
        body,
        grid=(grid,),
        in_specs=[pl.BlockSpec((TB2, 2 * D), lambda i: (i, 0)),
                  pl.BlockSpec((TB2, 2 * D), lambda i: (i, 0))]
                 + [ng_spec(n) for n in range(NEG)],
        out_specs=pl.BlockSpec(memory_space=pltpu.SMEM),
        out_shape=jax.ShapeDtypeStruct((1, 1), jnp.float32),
        compiler_params=pltpu.CompilerParams(
            dimension_semantics=("arbitrary",)),
    )(eu2, evng2, *([evng2] * NEG))
    return out[0, 0]


def kernel(pos_u, pos_v, neg_v, batch_size, u_weight, v_weight):
    B, P_ = pos_u.shape
    V = u_weight.shape[0]
    NEG = neg_v.shape[0] // B
    pu_t = pos_u.T.reshape(-1)
    pv_t = pos_v.T.reshape(-1)
    nv_t = neg_v.reshape(B, NEG, P_).transpose(1, 2, 0).reshape(-1)
    v_r = _tc_repack(v_weight.T, V)
    evng2 = _sc_gather_v(pv_t, nv_t, v_r, B, NEG)
    ut_dep, _ = lax.optimization_barrier((u_weight.T, v_r))
    u_r = _tc_repack(ut_dep, V)
    eu2 = _sc_gather_u(pu_t, u_r, B)
    total = _tc_loss(eu2, evng2, B, NEG)
    return -total / jnp.asarray(batch_size, jnp.float32)


# speedup vs baseline: 2.0645x; 1.1636x over previous
import jax
import jax.numpy as jnp
from jax import lax
from jax.experimental import pallas as pl
from jax.experimental.pallas import tpu as pltpu
from jax.experimental.pallas import tpu_sc as plsc

D = 64
PHRASE = 4
LANES = 16
C = 128
VC = 4096
HB = VC // 2


def _tc_repack(t, V):
    NB = (V + VC - 1) // VC
    NB2 = (NB + 1) // 2
    LB = (V + HB - 1) // HB - 1

    def spec(j):
        return pl.BlockSpec(
            (D, HB), lambda k, j=j: (0, jnp.minimum(4 * k + j, LB)))

    def body(s0, s1, s2, s3, dst):
        halves = []
        for src_t, src_b in ((s0, s1), (s2, s3)):
            tt = jnp.transpose(src_t[...], (1, 0))
            tb = jnp.transpose(src_b[...], (1, 0))
            halves.append(jnp.concatenate([tt, tb], axis=1))
        dst[...] = jnp.concatenate(halves, axis=0)

    return pl.pallas_call(
        body,
        grid=(NB2,),
        in_specs=[spec(0), spec(1), spec(2), spec(3)],
        out_specs=pl.BlockSpec((2 * HB, 2 * D), lambda k: (k, 0)),
        out_shape=jax.ShapeDtypeStruct((NB2 * 2 * HB, 2 * D), jnp.float32),
    )(t, t, t, t)


_SC_SCRATCH = (
    pltpu.VMEM((PHRASE, C), jnp.int32),
    pltpu.VMEM((PHRASE, C), jnp.int32),
    pltpu.VMEM((PHRASE * C, 2 * D), jnp.float32),
    pltpu.VMEM((C // 2, 2 * D), jnp.float32),
    pltpu.SemaphoreType.DMA,
)


def _make_do_chunk(B, idx_s, row_s, rows_s, emb_s, sem):
    def do_chunk(table_ref, idx_hbm, idx_row, src_base, out_ref, out_b2):
            for p in range(PHRASE):
                src = pl.multiple_of((idx_row + p) * B + src_base, 128)
                pltpu.sync_copy(idx_hbm.at[pl.ds(src, C)], idx_s.at[p])
            for p in range(PHRASE):
                for k in range(C // LANES):
                    v = idx_s[p, pl.ds(k * LANES, LANES)]
                    row_s[p, pl.ds(k * LANES, LANES)] = (
                        ((v >> 12) << 11) | (v & (HB - 1)))
            hs = [pltpu.async_copy(table_ref.at[row_s.at[p]],
                                   rows_s.at[pl.ds(p * C, C)], sem)
                  for p in range(PHRASE)]
            for h in hs:
                h.wait()

            def pool(g, carry):
                offv = [((idx_s[p, pl.ds(g * LANES, LANES)] >> 5) & 64)
                        for p in range(PHRASE)]
                for l in range(LANES):
                    i = g * LANES + l
                    i2 = g * (LANES // 2) + l // 2
                    e = l % 2
                    off = [offv[p][l] for p in range(PHRASE)]
                    for d in range(D // LANES):
                        acc = (rows_s[i, pl.ds(off[0] + d * LANES, LANES)]
                               + rows_s[C + i, pl.ds(off[1] + d * LANES, LANES)]
                               + rows_s[2 * C + i, pl.ds(off[2] + d * LANES, LANES)]
                               + rows_s[3 * C + i, pl.ds(off[3] + d * LANES, LANES)])
                        emb_s[i2, pl.ds(e * D + d * LANES, LANES)] = acc
                return carry
            lax.fori_loop(0, C // LANES, pool, 0)
            pltpu.sync_copy(emb_s,
                            out_ref.at[pl.ds(pl.multiple_of(out_b2, 64),
                                             C // 2)])

    return do_chunk


def _sc_mesh_info():
    info = plsc.get_sparse_core_info()
    return info.num_cores, info.num_subcores


def _sc_gather_u(pu_t, u_r, B):
    NC, NS = _sc_mesh_info()
    SB = B // (NC * NS)
    n_chunks = SB // C
    mesh = plsc.VectorSubcoreMesh(core_axis_name="c", subcore_axis_name="s")

    def body(pu_ref, u_ref, eu_ref, idx_s, row_s, rows_s, emb_s, sem):
        wid = lax.axis_index("s") * NC + lax.axis_index("c")
        base = wid * SB
        do_chunk = _make_do_chunk(B, idx_s, row_s, rows_s, emb_s, sem)

        def u_pass(c, carry):
            do_chunk(u_ref, pu_ref, 0, base + c * C, eu_ref,
                     (base + c * C) // 2)
            return carry
        lax.fori_loop(0, n_chunks, u_pass, 0)

    f = pl.kernel(
        body,
        out_type=jax.ShapeDtypeStruct((B // 2, 2 * D), jnp.float32),
        mesh=mesh,
        scratch_types=_SC_SCRATCH,
        compiler_params=pltpu.CompilerParams(use_tc_tiling_on_sc=True),
    )
    return f(pu_t, u_r)


def _sc_gather_v(pv_t, nv_t, v_r, B, NEG):
    NC, NS = _sc_mesh_info()
    SB = B // (NC * NS)
    n_chunks = SB // C
    mesh = plsc.VectorSubcoreMesh(core_axis_name="c", subcore_axis_name="s")

    def body(pv_ref, nv_ref, v_ref, evng_ref, idx_s, row_s, rows_s, emb_s,
             sem):
        wid = lax.axis_index("s") * NC + lax.axis_index("c")
        base = wid * SB
        do_chunk = _make_do_chunk(B, idx_s, row_s, rows_s, emb_s, sem)

        def v_pass(t, carry):
            c = t % n_chunks
            g = t // n_chunks
            src_base = base + c * C
            out_b2 = (g * B + base + c * C) // 2

            @pl.when(g == 0)
            def _():
                do_chunk(v_ref, pv_ref, 0, src_base, evng_ref, out_b2)

            @pl.when(g > 0)
            def _():
                do_chunk(v_ref, nv_ref, (g - 1) * PHRASE, src_base,
                         evng_ref, out_b2)
            return carry
        lax.fori_loop(0, (1 + NEG) * n_chunks, v_pass, 0)

    f = pl.kernel(
        body,
        out_type=jax.ShapeDtypeStruct(((1 + NEG) * B // 2, 2 * D),
                                      jnp.float32),
        mesh=mesh,
        scratch_types=_SC_SCRATCH,
        compiler_params=pltpu.CompilerParams(use_tc_tiling_on_sc=True),
    )
    return f(pv_t, nv_t, v_r)


def _tc_loss(eu2, evng2, B, NEG):
    TB2 = 512
    grid = (B // 2) // TB2
    nblk = (B // 2) // TB2
    inv = 1.0 / (PHRASE * PHRASE)

    def body(eu_ref, ev_ref, *rest):
        ng_refs, out_ref = rest[:-1], rest[-1]
        i = pl.program_id(0)
        eu = eu_ref[...]
        ev = ev_ref[...]

        def lsig(x):
            return jnp.minimum(x, 0.0) - jnp.log1p(jnp.exp(-jnp.abs(x)))

        def pair_scores(a, b):
            s0 = jnp.sum(a[:, :D] * b[:, :D], axis=1) * inv
            s1 = jnp.sum(a[:, D:] * b[:, D:], axis=1) * inv
            return s0, s1

        s0, s1 = pair_scores(eu, ev)
        tot = jnp.sum(lsig(s0)) + jnp.sum(lsig(s1))
        for ng_ref in ng_refs:
            n0, n1 = pair_scores(eu, ng_ref[...])
            tot = tot + jnp.sum(lsig(-n0)) + jnp.sum(lsig(-n1))

        @pl.when(i == 0)
        def _():
            out_ref[0, 0] = 0.0
        out_ref[0, 0] += tot

    def ng_spec(n):
        return pl.BlockSpec((TB2, 2 * D), lambda i, n=n: (nblk * (1 + n) + i, 0))

    out = pl.pallas_call(
        body,
        grid=(grid,),
        in_specs=[pl.BlockSpec((TB2, 2 * D), lambda i: (i, 0)),
                  pl.BlockSpec((TB2, 2 * D), lambda i: (i, 0))]
                 + [ng_spec(n) for n in range(NEG)],
        out_specs=pl.BlockSpec(memory_space=pltpu.SMEM),
        out_shape=jax.ShapeDtypeStruct((1, 1), jnp.float32),
        compiler_params=pltpu.CompilerParams(
            dimension_semantics=("arbitrary",)),
    )(eu2, evng2, *([evng2] * NEG))
    return out[0, 0]


def kernel(pos_u, pos_v, neg_v, batch_size, u_weight, v_weight):
    B, P_ = pos_u.shape
    V = u_weight.shape[0]
    NEG = neg_v.shape[0] // B
    pu_t = pos_u.T.reshape(-1)
    pv_t = pos_v.T.reshape(-1)
    nv_t = neg_v.reshape(B, NEG, P_).transpose(1, 2, 0).reshape(-1)
    v_r = _tc_repack(v_weight.T, V)
    evng2 = _sc_gather_v(pv_t, nv_t, v_r, B, NEG)
    ut_dep, _ = lax.optimization_barrier((u_weight.T, v_r))
    u_r = _tc_repack(ut_dep, V)
    eu2 = _sc_gather_u(pu_t, u_r, B)
    total = _tc_loss(eu2, evng2, B, NEG)
    return -total / jnp.asarray(batch_size, jnp.float32)


# speedup vs baseline: 2.1397x; 1.0364x over previous
import jax
import jax.numpy as jnp
from jax import lax
from jax.experimental import pallas as pl
from jax.experimental.pallas import tpu as pltpu
from jax.experimental.pallas import tpu_sc as plsc

D = 64
PHRASE = 4
LANES = 16
C = 128
VC = 4096
HB = VC // 2


def _tc_repack(t, V):
    PB = 4
    NB = (V + VC - 1) // VC
    NBS = (NB + PB - 1) // PB
    LB = (V + HB - 1) // HB - 1

    def spec(j):
        return pl.BlockSpec(
            (D, HB), lambda k, j=j: (0, jnp.minimum(2 * PB * k + j, LB)))

    def body(*refs):
        srcs, dst = refs[:-1], refs[-1]
        halves = []
        for b in range(PB):
            tt = jnp.transpose(srcs[2 * b][...], (1, 0))
            tb = jnp.transpose(srcs[2 * b + 1][...], (1, 0))
            halves.append(jnp.concatenate([tt, tb], axis=1))
        dst[...] = jnp.concatenate(halves, axis=0)

    return pl.pallas_call(
        body,
        grid=(NBS,),
        in_specs=[spec(j) for j in range(2 * PB)],
        out_specs=pl.BlockSpec((PB * HB, 2 * D), lambda k: (k, 0)),
        out_shape=jax.ShapeDtypeStruct((NBS * PB * HB, 2 * D), jnp.float32),
    )(*([t] * (2 * PB)))


_SC_SCRATCH = (
    pltpu.VMEM((PHRASE, C), jnp.int32),
    pltpu.VMEM((PHRASE, C), jnp.int32),
    pltpu.VMEM((PHRASE * C, 2 * D), jnp.float32),
    pltpu.VMEM((C // 2, 2 * D), jnp.float32),
    pltpu.SemaphoreType.DMA,
)


def _make_do_chunk(B, idx_s, row_s, rows_s, emb_s, sem):
    def do_chunk(table_ref, idx_hbm, idx_row, src_base, out_ref, out_b2):
            for p in range(PHRASE):
                src = pl.multiple_of((idx_row + p) * B + src_base, 128)
                pltpu.sync_copy(idx_hbm.at[pl.ds(src, C)], idx_s.at[p])
            for p in range(PHRASE):
                for k in range(C // LANES):
                    v = idx_s[p, pl.ds(k * LANES, LANES)]
                    row_s[p, pl.ds(k * LANES, LANES)] = (
                        ((v >> 12) << 11) | (v & (HB - 1)))
            hs = [pltpu.async_copy(table_ref.at[row_s.at[p]],
                                   rows_s.at[pl.ds(p * C, C)], sem)
                  for p in range(PHRASE)]
            for h in hs:
                h.wait()

            def pool(g, carry):
                offv = [((idx_s[p, pl.ds(g * LANES, LANES)] >> 5) & 64)
                        for p in range(PHRASE)]
                for l in range(LANES):
                    i = g * LANES + l
                    i2 = g * (LANES // 2) + l // 2
                    e = l % 2
                    off = [offv[p][l] for p in range(PHRASE)]
                    for d in range(D // LANES):
                        acc = (rows_s[i, pl.ds(off[0] + d * LANES, LANES)]
                               + rows_s[C + i, pl.ds(off[1] + d * LANES, LANES)]
                               + rows_s[2 * C + i, pl.ds(off[2] + d * LANES, LANES)]
                               + rows_s[3 * C + i, pl.ds(off[3] + d * LANES, LANES)])
                        emb_s[i2, pl.ds(e * D + d * LANES, LANES)] = acc
                return carry
            lax.fori_loop(0, C // LANES, pool, 0)
            pltpu.sync_copy(emb_s,
                            out_ref.at[pl.ds(pl.multiple_of(out_b2, 64),
                                             C // 2)])

    return do_chunk


def _sc_mesh_info():
    info = plsc.get_sparse_core_info()
    return info.num_cores, info.num_subcores


def _sc_gather_u(pu_t, u_r, B):
    NC, NS = _sc_mesh_info()
    SB = B // (NC * NS)
    n_chunks = SB // C
    mesh = plsc.VectorSubcoreMesh(core_axis_name="c", subcore_axis_name="s")

    def body(pu_ref, u_ref, eu_ref, idx_s, row_s, rows_s, emb_s, sem):
        wid = lax.axis_index("s") * NC + lax.axis_index("c")
        base = wid * SB
        do_chunk = _make_do_chunk(B, idx_s, row_s, rows_s, emb_s, sem)

        def u_pass(c, carry):
            do_chunk(u_ref, pu_ref, 0, base + c * C, eu_ref,
                     (base + c * C) // 2)
            return carry
        lax.fori_loop(0, n_chunks, u_pass, 0)

    f = pl.kernel(
        body,
        out_type=jax.ShapeDtypeStruct((B // 2, 2 * D), jnp.float32),
        mesh=mesh,
        scratch_types=_SC_SCRATCH,
        compiler_params=pltpu.CompilerParams(use_tc_tiling_on_sc=True),
    )
    return f(pu_t, u_r)


def _sc_gather_v(pv_t, nv_t, v_r, B, NEG):
    NC, NS = _sc_mesh_info()
    SB = B // (NC * NS)
    n_chunks = SB // C
    mesh = plsc.VectorSubcoreMesh(core_axis_name="c", subcore_axis_name="s")

    def body(pv_ref, nv_ref, v_ref, evng_ref, idx_s, row_s, rows_s, emb_s,
             sem):
        wid = lax.axis_index("s") * NC + lax.axis_index("c")
        base = wid * SB
        do_chunk = _make_do_chunk(B, idx_s, row_s, rows_s, emb_s, sem)

        def v_pass(t, carry):
            c = t % n_chunks
            g = t // n_chunks
            src_base = base + c * C
            out_b2 = (g * B + base + c * C) // 2

            @pl.when(g == 0)
            def _():
                do_chunk(v_ref, pv_ref, 0, src_base, evng_ref, out_b2)

            @pl.when(g > 0)
            def _():
                do_chunk(v_ref, nv_ref, (g - 1) * PHRASE, src_base,
                         evng_ref, out_b2)
            return carry
        lax.fori_loop(0, (1 + NEG) * n_chunks, v_pass, 0)

    f = pl.kernel(
        body,
        out_type=jax.ShapeDtypeStruct(((1 + NEG) * B // 2, 2 * D),
                                      jnp.float32),
        mesh=mesh,
        scratch_types=_SC_SCRATCH,
        compiler_params=pltpu.CompilerParams(use_tc_tiling_on_sc=True),
    )
    return f(pv_t, nv_t, v_r)


def _tc_loss(eu2, evng2, B, NEG):
    TB2 = 512
    grid = (B // 2) // TB2
    nblk = (B // 2) // TB2
    inv = 1.0 / (PHRASE * PHRASE)

    def body(eu_ref, ev_ref, *rest):
        ng_refs, out_ref = rest[:-1], rest[-1]
        i = pl.program_id(0)
        eu = eu_ref[...]
        ev = ev_ref[...]

        def lsig(x):
            return jnp.minimum(x, 0.0) - jnp.log1p(jnp.exp(-jnp.abs(x)))

        def pair_scores(a, b):
            s0 = jnp.sum(a[:, :D] * b[:, :D], axis=1) * inv
            s1 = jnp.sum(a[:, D:] * b[:, D:], axis=1) * inv
            return s0, s1

        s0, s1 = pair_scores(eu, ev)
        tot = jnp.sum(lsig(s0)) + jnp.sum(lsig(s1))
        for ng_ref in ng_refs:
            n0, n1 = pair_scores(eu, ng_ref[...])
            tot = tot + jnp.sum(lsig(-n0)) + jnp.sum(lsig(-n1))

        @pl.when(i == 0)
        def _():
            out_ref[0, 0] = 0.0
        out_ref[0, 0] += tot

    def ng_spec(n):
        return pl.BlockSpec((TB2, 2 * D), lambda i, n=n: (nblk * (1 + n) + i, 0))

    out = pl.pallas_call(
        body,
        grid=(grid,),
        in_specs=[pl.BlockSpec((TB2, 2 * D), lambda i: (i, 0)),
                  pl.BlockSpec((TB2, 2 * D), lambda i: (i, 0))]
                 + [ng_spec(n) for n in range(NEG)],
        out_specs=pl.BlockSpec(memory_space=pltpu.SMEM),
        out_shape=jax.ShapeDtypeStruct((1, 1), jnp.float32),
        compiler_params=pltpu.CompilerParams(
            dimension_semantics=("arbitrary",)),
    )(eu2, evng2, *([evng2] * NEG))
    return out[0, 0]


def kernel(pos_u, pos_v, neg_v, batch_size, u_weight, v_weight):
    B, P_ = pos_u.shape
    V = u_weight.shape[0]
    NEG = neg_v.shape[0] // B
    pu_t = pos_u.T.reshape(-1)
    pv_t = pos_v.T.reshape(-1)
    nv_t = neg_v.reshape(B, NEG, P_).transpose(1, 2, 0).reshape(-1)
    v_r = _tc_repack(v_weight.T, V)
    evng2 = _sc_gather_v(pv_t, nv_t, v_r, B, NEG)
    ut_dep, _ = lax.optimization_barrier((u_weight.T, v_r))
    u_r = _tc_repack(ut_dep, V)
    eu2 = _sc_gather_u(pu_t, u_r, B)
    total = _tc_loss(eu2, evng2, B, NEG)
    return -total / jnp.asarray(batch_size, jnp.float32)


# speedup vs baseline: 2.2353x; 1.0447x over previous
import jax
import jax.numpy as jnp
from jax import lax
from jax.experimental import pallas as pl
from jax.experimental.pallas import tpu as pltpu
from jax.experimental.pallas import tpu_sc as plsc

D = 64
PHRASE = 4
LANES = 16
C = 64
VC = 4096
HB = VC // 2


def _tc_repack(t, V):
    PB = 4
    NB = (V + VC - 1) // VC
    NBS = (NB + PB - 1) // PB
    LB = (V + HB - 1) // HB - 1

    def spec(j):
        return pl.BlockSpec(
            (D, HB), lambda k, j=j: (0, jnp.minimum(2 * PB * k + j, LB)))

    def body(*refs):
        srcs, dst = refs[:-1], refs[-1]
        halves = []
        for b in range(PB):
            tt = jnp.transpose(srcs[2 * b][...], (1, 0))
            tb = jnp.transpose(srcs[2 * b + 1][...], (1, 0))
            halves.append(jnp.concatenate([tt, tb], axis=1))
        dst[...] = jnp.concatenate(halves, axis=0)

    return pl.pallas_call(
        body,
        grid=(NBS,),
        in_specs=[spec(j) for j in range(2 * PB)],
        out_specs=pl.BlockSpec((PB * HB, 2 * D), lambda k: (k, 0)),
        out_shape=jax.ShapeDtypeStruct((NBS * PB * HB, 2 * D), jnp.float32),
    )(*([t] * (2 * PB)))


_SC_SCRATCH = (
    pltpu.VMEM((2, PHRASE, C), jnp.int32),
    pltpu.VMEM((2, PHRASE, C), jnp.int32),
    pltpu.VMEM((2, PHRASE * C, 2 * D), jnp.float32),
    pltpu.VMEM((2, C // 2, 2 * D), jnp.float32),
    pltpu.SemaphoreType.DMA,
    pltpu.SemaphoreType.DMA,
)


def _make_pipeline(B, idx_s, row_s, rows_s, emb_s, sems):
    def stage(b, table_ref, idx_hbm, idx_row, src_base):
        for p in range(PHRASE):
            src = pl.multiple_of((idx_row + p) * B + src_base, C)
            pltpu.sync_copy(idx_hbm.at[pl.ds(src, C)], idx_s.at[b, p])
        for p in range(PHRASE):
            for k in range(C // LANES):
                v = idx_s[b, p, pl.ds(k * LANES, LANES)]
                row_s[b, p, pl.ds(k * LANES, LANES)] = (
                    ((v >> 12) << 11) | (v & (HB - 1)))
        for p in range(PHRASE):
            pltpu.async_copy(table_ref.at[row_s.at[b, p]],
                             rows_s.at[b, pl.ds(p * C, C)], sems[b])

    def finish(b, table_ref, out_ref, out_b2):
        for p in range(PHRASE):
            pltpu.make_async_copy(table_ref.at[row_s.at[b, p]],
                                  rows_s.at[b, pl.ds(p * C, C)],
                                  sems[b]).wait()

        def pool(g, carry):
            offv = [((idx_s[b, p, pl.ds(g * LANES, LANES)] >> 5) & 64)
                    for p in range(PHRASE)]
            for l in range(LANES):
                i = g * LANES + l
                i2 = g * (LANES // 2) + l // 2
                e = l % 2
                off = [offv[p][l] for p in range(PHRASE)]
                for d in range(D // LANES):
                    acc = (rows_s[b, i, pl.ds(off[0] + d * LANES, LANES)]
                           + rows_s[b, C + i, pl.ds(off[1] + d * LANES, LANES)]
                           + rows_s[b, 2 * C + i, pl.ds(off[2] + d * LANES, LANES)]
                           + rows_s[b, 3 * C + i, pl.ds(off[3] + d * LANES, LANES)])
                    emb_s[b, i2, pl.ds(e * D + d * LANES, LANES)] = acc
            return carry
        lax.fori_loop(0, C // LANES, pool, 0)
        pltpu.sync_copy(emb_s.at[b],
                        out_ref.at[pl.ds(pl.multiple_of(out_b2, C // 2),
                                         C // 2)])

    return stage, finish


def _sc_mesh_info():
    info = plsc.get_sparse_core_info()
    return info.num_cores, info.num_subcores


def _sc_gather_u(pu_t, u_r, B):
    NC, NS = _sc_mesh_info()
    SB = B // (NC * NS)
    n_chunks = SB // C
    mesh = plsc.VectorSubcoreMesh(core_axis_name="c", subcore_axis_name="s")

    def body(pu_ref, u_ref, eu_ref, idx_s, row_s, rows_s, emb_s, s0, s1):
        wid = lax.axis_index("s") * NC + lax.axis_index("c")
        base = wid * SB
        stage, finish = _make_pipeline(B, idx_s, row_s, rows_s, emb_s,
                                       (s0, s1))
        stage(0, u_ref, pu_ref, 0, base)

        def step(t, b):
            @pl.when(t + 1 < n_chunks)
            def _():
                stage(1 - b, u_ref, pu_ref, 0, base + (t + 1) * C)
            finish(b, u_ref, eu_ref, (base + t * C) // 2)

        def loop_body(t, carry):
            @pl.when(t % 2 == 0)
            def _():
                step(t, 0)

            @pl.when(t % 2 == 1)
            def _():
                step(t, 1)
            return carry
        lax.fori_loop(0, n_chunks, loop_body, 0)

    f = pl.kernel(
        body,
        out_type=jax.ShapeDtypeStruct((B // 2, 2 * D), jnp.float32),
        mesh=mesh,
        scratch_types=_SC_SCRATCH,
        compiler_params=pltpu.CompilerParams(use_tc_tiling_on_sc=True),
    )
    return f(pu_t, u_r)


def _sc_gather_v(pvnv_t, v_r, B, NEG):
    NC, NS = _sc_mesh_info()
    SB = B // (NC * NS)
    n_chunks = SB // C
    T = (1 + NEG) * n_chunks
    mesh = plsc.VectorSubcoreMesh(core_axis_name="c", subcore_axis_name="s")

    def body(pvnv_ref, v_ref, evng_ref, idx_s, row_s, rows_s, emb_s, s0, s1):
        wid = lax.axis_index("s") * NC + lax.axis_index("c")
        base = wid * SB

        def coords(t):
            c = t % n_chunks
            g = t // n_chunks
            return g * PHRASE, base + c * C, (g * B + base + c * C) // 2

        stage, finish = _make_pipeline(B, idx_s, row_s, rows_s, emb_s,
                                       (s0, s1))
        r0, sb0, _ = coords(0)
        stage(0, v_ref, pvnv_ref, r0, sb0)

        def step(t, b):
            @pl.when(t + 1 < T)
            def _():
                r, sb, _ = coords(t + 1)
                stage(1 - b, v_ref, pvnv_ref, r, sb)
            _, _, ob2 = coords(t)
            finish(b, v_ref, evng_ref, ob2)

        def loop_body(t, carry):
            @pl.when(t % 2 == 0)
            def _():
                step(t, 0)

            @pl.when(t % 2 == 1)
            def _():
                step(t, 1)
            return carry
        lax.fori_loop(0, T, loop_body, 0)

    f = pl.kernel(
        body,
        out_type=jax.ShapeDtypeStruct(((1 + NEG) * B // 2, 2 * D),
                                      jnp.float32),
        mesh=mesh,
        scratch_types=_SC_SCRATCH,
        compiler_params=pltpu.CompilerParams(use_tc_tiling_on_sc=True),
    )
    return f(pvnv_t, v_r)


def _tc_loss(eu2, evng2, B, NEG):
    TB2 = 512
    grid = (B // 2) // TB2
    nblk = (B // 2) // TB2
    inv = 1.0 / (PHRASE * PHRASE)

    def body(eu_ref, ev_ref, *rest):
        ng_refs, out_ref = rest[:-1], rest[-1]
        i = pl.program_id(0)
        eu = eu_ref[...]
        ev = ev_ref[...]

        def lsig(x):
            return jnp.minimum(x, 0.0) - jnp.log1p(jnp.exp(-jnp.abs(x)))

        def pair_scores(a, b):
            s0 = jnp.sum(a[:, :D] * b[:, :D], axis=1) * inv
            s1 = jnp.sum(a[:, D:] * b[:, D:], axis=1) * inv
            return s0, s1

        s0, s1 = pair_scores(eu, ev)
        tot = jnp.sum(lsig(s0)) + jnp.sum(lsig(s1))
        for ng_ref in ng_refs:
            n0, n1 = pair_scores(eu, ng_ref[...])
            tot = tot + jnp.sum(lsig(-n0)) + jnp.sum(lsig(-n1))

        @pl.when(i == 0)
        def _():
            out_ref[0, 0] = 0.0
        out_ref[0, 0] += tot

    def ng_spec(n):
        return pl.BlockSpec((TB2, 2 * D), lambda i, n=n: (nblk * (1 + n) + i, 0))

    out = pl.pallas_call(
        body,
        grid=(grid,),
        in_specs=[pl.BlockSpec((TB2, 2 * D), lambda i: (i, 0)),
                  pl.BlockSpec((TB2, 2 * D), lambda i: (i, 0))]
                 + [ng_spec(n) for n in range(NEG)],
        out_specs=pl.BlockSpec(memory_space=pltpu.SMEM),
        out_shape=jax.ShapeDtypeStruct((1, 1), jnp.float32),
        compiler_params=pltpu.CompilerParams(
            dimension_semantics=("arbitrary",)),
    )(eu2, evng2, *([evng2] * NEG))
    return out[0, 0]


def kernel(pos_u, pos_v, neg_v, batch_size, u_weight, v_weight):
    B, P_ = pos_u.shape
    V = u_weight.shape[0]
    NEG = neg_v.shape[0] // B
    pu_t = pos_u.T.reshape(-1)
    pvnv_t = jnp.concatenate(
        [pos_v.T.reshape(-1),
         neg_v.reshape(B, NEG, P_).transpose(1, 2, 0).reshape(-1)])
    v_r = _tc_repack(v_weight.T, V)
    evng2 = _sc_gather_v(pvnv_t, v_r, B, NEG)
    ut_dep, _ = lax.optimization_barrier((u_weight.T, v_r))
    u_r = _tc_repack(ut_dep, V)
    eu2 = _sc_gather_u(pu_t, u_r, B)
    total = _tc_loss(eu2, evng2, B, NEG)
    return -total / jnp.asarray(batch_size, jnp.float32)
